# baseline (speedup 1.0000x reference)
import functools
import math

import jax
import jax.numpy as jnp
import numpy as np
from jax import lax
from jax.experimental import pallas as pl
from jax.experimental.pallas import tpu as pltpu
from jax.experimental.pallas import tpu_sc as plsc

_MINUTE_SIZE = 4
_HOUR_SIZE = 24

_NC = 2
_NS = 16
_NW = _NC * _NS
_NB = 4


def _pos_table(emb_size, maxlen):
    den = np.exp(-np.arange(0, emb_size, 2, dtype=np.float64) * math.log(10000.0) / emb_size)
    pos = np.arange(0, maxlen, dtype=np.float64).reshape(maxlen, 1)
    pe = np.zeros((maxlen, emb_size), dtype=np.float32)
    pe[:, 0::2] = np.sin(pos * den).astype(np.float32)
    pe[:, 1::2] = np.cos(pos * den).astype(np.float32)
    return jnp.asarray(pe)


def kernel(src, mode, time, weekday, emb_loc, emb_mode, minute_embed, hour_embed, weekday_embed):
    S, B = src.shape
    V, D = emb_loc.shape
    scale = math.sqrt(D)
    n_wk = weekday_embed.shape[0]

    mw = ((emb_mode[:, None, :] + weekday_embed[None, :, :]) * scale).reshape(-1, D)
    tt = ((jnp.repeat(hour_embed, _MINUTE_SIZE, axis=0)
           + jnp.tile(minute_embed, (_HOUR_SIZE, 1))) * scale)

    pe = _pos_table(D, S)

    CB = B // _NW
    NG = CB // 16
    NJ = D // 16
    n_mw = mw.shape[0]
    n_tt = tt.shape[0]

    mesh = plsc.VectorSubcoreMesh(core_axis_name="c", subcore_axis_name="s",
                                  num_cores=_NC, num_subcores=_NS)

    @functools.partial(
        pl.kernel,
        out_type=jax.ShapeDtypeStruct((S, B, D), jnp.float32),
        mesh=mesh,
        compiler_params=pltpu.CompilerParams(use_tc_tiling_on_sc=False),
        scratch_types=[
            pltpu.VMEM((_NB, CB), jnp.int32),
            pltpu.VMEM((_NB, CB), jnp.int32),
            pltpu.VMEM((_NB, CB), jnp.int32),
            pltpu.VMEM((_NB, CB), jnp.int32),
            pltpu.VMEM((_NB, CB), jnp.int32),
            pltpu.VMEM((_NB, CB, D), jnp.float32),
            pltpu.VMEM((S, D), jnp.float32),
            pltpu.VMEM((n_mw, D), jnp.float32),
            pltpu.VMEM((n_tt, D), jnp.float32),
            pltpu.SemaphoreType.DMA((_NB,)),
            pltpu.SemaphoreType.DMA((_NB,)),
            pltpu.SemaphoreType.DMA((_NB,)),
        ],
    )
    def _sc_kernel(src_h, mode_h, time_h, wk_h, loc_h, mw_h, tt_h, pe_h, out_h,
                   src_v, mode_v, time_v, wk_v, cim_v, rows_v, pe_all, mw_vm, tt_vm,
                   sem_idx, sem_g, sem_out):
        wid = lax.axis_index("s") * _NC + lax.axis_index("c")
        base = wid * CB
        arrs = [(src_h, src_v), (mode_h, mode_v), (time_h, time_v), (wk_h, wk_v)]
        lanes = [lax.iota(jnp.int32, 16) + (j * 16) for j in range(NJ)]

        def issue_idx(g, b):
            for h, v in arrs:
                pltpu.async_copy(h.at[g, pl.ds(base, CB)], v.at[b], sem_idx.at[b])

        def wait_idx(g, b):
            for h, v in arrs:
                pltpu.make_async_copy(h.at[g, pl.ds(base, CB)], v.at[b], sem_idx.at[b]).wait()

        def compute_ci(b):
            for gg in range(NG):
                sl = pl.ds(gg * 16, 16)
                cim_v[b, sl] = mode_v[b, sl] * n_wk + wk_v[b, sl]

        def issue_gather(b):
            pltpu.async_copy(loc_h.at[src_v.at[b]], rows_v.at[b], sem_g.at[b])

        def wait_gather(b):
            pltpu.make_async_copy(loc_h.at[src_v.at[b]], rows_v.at[b], sem_g.at[b]).wait()

        def issue_out(g, b):
            pltpu.async_copy(rows_v.at[b], out_h.at[g, pl.ds(base, CB)], sem_out.at[b])

        def wait_out(g, b):
            pltpu.make_async_copy(rows_v.at[b], out_h.at[g, pl.ds(base, CB)], sem_out.at[b]).wait()

        def compute(g, b):
            pe_regs = [pe_all[g, pl.ds(j * 16, 16)] for j in range(NJ)]

            @plsc.parallel_loop(0, NG, step=1, unroll=1)
            def grp_body(gg):
                i0 = gg * 16
                cmv = cim_v[b, pl.ds(i0, 16)]
                ctv = time_v[b, pl.ds(i0, 16)]
                sls = [pl.ds(j * 16, 16) for j in range(NJ)]
                for k in range(16):
                    i = i0 + k
                    cm = cmv[k]
                    ct = ctv[k]
                    r = [rows_v[b, i, sl] for sl in sls]
                    sm = [mw_vm[cm, sl] + tt_vm[ct, sl] for sl in sls]
                    for j, sl in enumerate(sls):
                        rows_v[b, i, sl] = r[j] * scale + (sm[j] + pe_regs[j])

        pltpu.sync_copy(pe_h, pe_all)
        pltpu.sync_copy(mw_h, mw_vm)
        pltpu.sync_copy(tt_h, tt_vm)
        for b in range(_NB):
            issue_idx(b, b)
        for b in range(_NB):
            wait_idx(b, b)
            compute_ci(b)
            issue_gather(b)

        def body(G, c):
            for b in range(_NB):
                g = G * _NB + b
                wait_gather(b)
                compute(g, b)

                @pl.when(g + _NB < S)
                def _():
                    issue_idx(g + _NB, b)

                issue_out(g, b)
            for b in range(_NB):
                g = G * _NB + b

                @pl.when(g + _NB < S)
                def _():
                    wait_idx(g + _NB, b)
                    compute_ci(b)
                    wait_out(g, b)
                    issue_gather(b)

            return c

        lax.fori_loop(0, S // _NB, body, 0)
        for b in range(_NB):
            wait_out(S - _NB + b, b)

    return _sc_kernel(src, mode, time, weekday, emb_loc, mw, tt, pe)


# speedup vs baseline: 1.1025x; 1.1025x over previous
import functools
import math

import jax
import jax.numpy as jnp
import numpy as np
from jax import lax
from jax.experimental import pallas as pl
from jax.experimental.pallas import tpu as pltpu
from jax.experimental.pallas import tpu_sc as plsc

_MINUTE_SIZE = 4
_HOUR_SIZE = 24

_NC = 2
_NS = 16
_NW = _NC * _NS
_NB = 4


def _pos_table(emb_size, maxlen):
    den = np.exp(-np.arange(0, emb_size, 2, dtype=np.float64) * math.log(10000.0) / emb_size)
    pos = np.arange(0, maxlen, dtype=np.float64).reshape(maxlen, 1)
    pe = np.zeros((maxlen, emb_size), dtype=np.float32)
    pe[:, 0::2] = np.sin(pos * den).astype(np.float32)
    pe[:, 1::2] = np.cos(pos * den).astype(np.float32)
    return jnp.asarray(pe)


def kernel(src, mode, time, weekday, emb_loc, emb_mode, minute_embed, hour_embed, weekday_embed):
    S, B = src.shape
    V, D = emb_loc.shape
    scale = math.sqrt(D)
    n_time = _MINUTE_SIZE * _HOUR_SIZE
    n_wk = weekday_embed.shape[0]

    t_time = (jnp.repeat(hour_embed, _MINUTE_SIZE, axis=0)
              + jnp.tile(minute_embed, (_HOUR_SIZE, 1)))
    t2 = (emb_mode[:, None, None, :] + t_time[None, :, None, :]
          + weekday_embed[None, None, :, :]) * scale
    t2 = t2.reshape(-1, D)

    pe = _pos_table(D, S)

    CB = B // _NW
    NG = CB // 16
    NJ = D // 16

    mesh = plsc.VectorSubcoreMesh(core_axis_name="c", subcore_axis_name="s",
                                  num_cores=_NC, num_subcores=_NS)

    @functools.partial(
        pl.kernel,
        out_type=jax.ShapeDtypeStruct((S * B * D // 128, 128), jnp.float32),
        mesh=mesh,
        compiler_params=pltpu.CompilerParams(use_tc_tiling_on_sc=False),
        scratch_types=[
            pltpu.VMEM((_NB, CB), jnp.int32),
            pltpu.VMEM((_NB, CB), jnp.int32),
            pltpu.VMEM((_NB, CB), jnp.int32),
            pltpu.VMEM((_NB, CB), jnp.int32),
            pltpu.VMEM((_NB, CB), jnp.int32),
            pltpu.VMEM((_NB, CB, D), jnp.float32),
            pltpu.VMEM((_NB, CB, D), jnp.float32),
            pltpu.VMEM((_NB, CB // 2, 2 * D), jnp.float32),
            pltpu.VMEM((S, D), jnp.float32),
            pltpu.SemaphoreType.DMA((_NB,)),
            pltpu.SemaphoreType.DMA((_NB,)),
            pltpu.SemaphoreType.DMA((_NB,)),
        ],
    )
    def _sc_kernel(src_h, mode_h, time_h, wk_h, loc_h, t2_h, pe_h, out_h,
                   src_v, mode_v, time_v, wk_v, ci_v, rows_v, small_v, res_v, pe_all,
                   sem_idx, sem_g, sem_out):
        wid = lax.axis_index("s") * _NC + lax.axis_index("c")
        base = wid * CB
        arrs = [(src_h, src_v), (mode_h, mode_v), (time_h, time_v), (wk_h, wk_v)]

        def issue_idx(g, b):
            for h, v in arrs:
                pltpu.async_copy(h.at[g, pl.ds(base, CB)], v.at[b], sem_idx.at[b])

        def wait_idx(g, b):
            for h, v in arrs:
                pltpu.make_async_copy(h.at[g, pl.ds(base, CB)], v.at[b], sem_idx.at[b]).wait()

        def compute_ci(b):
            for gg in range(NG):
                sl = pl.ds(gg * 16, 16)
                ci_v[b, sl] = (mode_v[b, sl] * n_time + time_v[b, sl]) * n_wk + wk_v[b, sl]

        def issue_gather(b):
            pltpu.async_copy(loc_h.at[src_v.at[b]], rows_v.at[b], sem_g.at[b])
            pltpu.async_copy(t2_h.at[ci_v.at[b]], small_v.at[b], sem_g.at[b])

        def wait_gather(b):
            pltpu.make_async_copy(loc_h.at[src_v.at[b]], rows_v.at[b], sem_g.at[b]).wait()
            pltpu.make_async_copy(t2_h.at[ci_v.at[b]], small_v.at[b], sem_g.at[b]).wait()

        def issue_out(g, b):
            pltpu.async_copy(res_v.at[b], out_h.at[pl.ds((g * B + base) // 2, CB // 2)], sem_out.at[b])

        def wait_out(g, b):
            pltpu.make_async_copy(res_v.at[b], out_h.at[pl.ds((g * B + base) // 2, CB // 2)], sem_out.at[b]).wait()

        def compute(g, b):
            pe_regs = [pe_all[g, pl.ds(j * 16, 16)] for j in range(NJ)]

            @plsc.parallel_loop(0, CB // 2, step=1, unroll=2)
            def pair_body(p):
                sls = [pl.ds(j * 16, 16) for j in range(NJ)]
                for t in range(2):
                    i = 2 * p + t
                    r = [rows_v[b, i, sl] for sl in sls]
                    sm = [small_v[b, i, sl] for sl in sls]
                    for j in range(NJ):
                        res_v[b, p, pl.ds(t * D + j * 16, 16)] = (
                            r[j] * scale + (sm[j] + pe_regs[j]))

        pltpu.sync_copy(pe_h, pe_all)
        for b in range(_NB):
            issue_idx(b, b)
        for b in range(_NB):
            wait_idx(b, b)
            compute_ci(b)
            issue_gather(b)

        def body(G, c):
            for b in range(_NB):
                g = G * _NB + b
                wait_gather(b)

                @pl.when(g + _NB < S)
                def _():
                    issue_idx(g + _NB, b)

                compute(g, b)
                issue_out(g, b)
            for b in range(_NB):
                g = G * _NB + b

                @pl.when(g + _NB < S)
                def _():
                    wait_idx(g + _NB, b)
                    compute_ci(b)
                    wait_out(g, b)
                    issue_gather(b)

            return c

        lax.fori_loop(0, S // _NB, body, 0)
        for b in range(_NB):
            wait_out(S - _NB + b, b)

    return _sc_kernel(src, mode, time, weekday, emb_loc, t2, pe).reshape(S, B, D)


# speedup vs baseline: 1.1316x; 1.0264x over previous
import functools
import math

import jax
import jax.numpy as jnp
import numpy as np
from jax import lax
from jax.experimental import pallas as pl
from jax.experimental.pallas import tpu as pltpu
from jax.experimental.pallas import tpu_sc as plsc

_MINUTE_SIZE = 4
_HOUR_SIZE = 24

_NC = 2
_NS = 16
_NW = _NC * _NS
_NB = 4
_LD = 3
_RB = 10


def _pos_table(emb_size, maxlen):
    den = np.exp(-np.arange(0, emb_size, 2, dtype=np.float64) * math.log(10000.0) / emb_size)
    pos = np.arange(0, maxlen, dtype=np.float64).reshape(maxlen, 1)
    pe = np.zeros((maxlen, emb_size), dtype=np.float32)
    pe[:, 0::2] = np.sin(pos * den).astype(np.float32)
    pe[:, 1::2] = np.cos(pos * den).astype(np.float32)
    return jnp.asarray(pe)


def kernel(src, mode, time, weekday, emb_loc, emb_mode, minute_embed, hour_embed, weekday_embed):
    S, B = src.shape
    V, D = emb_loc.shape
    scale = math.sqrt(D)
    n_time = _MINUTE_SIZE * _HOUR_SIZE
    n_wk = weekday_embed.shape[0]

    t_time = (jnp.repeat(hour_embed, _MINUTE_SIZE, axis=0)
              + jnp.tile(minute_embed, (_HOUR_SIZE, 1)))
    t2 = (emb_mode[:, None, None, :] + t_time[None, :, None, :]
          + weekday_embed[None, None, :, :]) * scale
    t2 = t2.reshape(-1, D)

    pe = _pos_table(D, S)

    CB = B // _NW
    NG = CB // 16
    NJ = D // 16
    RPB = 2 * _RB

    mesh = plsc.VectorSubcoreMesh(core_axis_name="c", subcore_axis_name="s",
                                  num_cores=_NC, num_subcores=_NS)

    @functools.partial(
        pl.kernel,
        out_type=jax.ShapeDtypeStruct((S, B, D), jnp.float32),
        mesh=mesh,
        compiler_params=pltpu.CompilerParams(use_tc_tiling_on_sc=False),
        scratch_types=[
            pltpu.VMEM((2, _RB, CB), jnp.int32),
            pltpu.VMEM((2, _RB, CB), jnp.int32),
            pltpu.VMEM((2, _RB, CB), jnp.int32),
            pltpu.VMEM((2, _RB, CB), jnp.int32),
            pltpu.VMEM((_NB, CB), jnp.int32),
            pltpu.VMEM((_NB, CB, D), jnp.float32),
            pltpu.VMEM((_NB, CB, D), jnp.float32),
            pltpu.VMEM((S, D), jnp.float32),
            pltpu.SemaphoreType.DMA((2,)),
            pltpu.SemaphoreType.DMA((_NB,)),
            pltpu.SemaphoreType.DMA((_NB,)),
        ],
    )
    def _sc_kernel(src_h, mode_h, time_h, wk_h, loc_h, t2_h, pe_h, out_h,
                   src_v, mode_v, time_v, wk_v, ci_v, rows_v, small_v, pe_all,
                   sem_idx, sem_g, sem_out):
        wid = lax.axis_index("s") * _NC + lax.axis_index("c")
        base = wid * CB
        arrs = [(src_h, src_v), (mode_h, mode_v), (time_h, time_v), (wk_h, wk_v)]

        def issue_idx(blk, kb):
            for h, v in arrs:
                pltpu.async_copy(h.at[pl.ds(blk * _RB, _RB), pl.ds(base, CB)],
                                 v.at[kb], sem_idx.at[kb])

        def wait_idx(blk, kb):
            for h, v in arrs:
                pltpu.make_async_copy(h.at[pl.ds(blk * _RB, _RB), pl.ds(base, CB)],
                                      v.at[kb], sem_idx.at[kb]).wait()

        def compute_ci(kb, r, b):
            for gg in range(NG):
                sl = pl.ds(gg * 16, 16)
                ci_v[b, sl] = (mode_v[kb, r, sl] * n_time + time_v[kb, r, sl]) * n_wk \
                    + wk_v[kb, r, sl]

        def issue_gather(kb, r, b):
            pltpu.async_copy(loc_h.at[src_v.at[kb, r]], rows_v.at[b], sem_g.at[b])
            pltpu.async_copy(t2_h.at[ci_v.at[b]], small_v.at[b], sem_g.at[b])

        def wait_gather(b):
            pltpu.make_async_copy(loc_h.at[src_v.at[0, 0]], rows_v.at[b], sem_g.at[b]).wait()
            pltpu.make_async_copy(t2_h.at[ci_v.at[b]], small_v.at[b], sem_g.at[b]).wait()

        def issue_out(g, b):
            pltpu.async_copy(rows_v.at[b], out_h.at[g, pl.ds(base, CB)], sem_out.at[b])

        def wait_out_b(b):
            pltpu.make_async_copy(rows_v.at[b], out_h.at[0, pl.ds(base, CB)], sem_out.at[b]).wait()

        def compute(g, b):
            pe_regs = [pe_all[g, pl.ds(j * 16, 16)] for j in range(NJ)]

            @plsc.parallel_loop(0, CB, step=1, unroll=4)
            def tok_body(i):
                sls = [pl.ds(j * 16, 16) for j in range(NJ)]
                r = [rows_v[b, i, sl] for sl in sls]
                sm = [small_v[b, i, sl] for sl in sls]
                for j, sl in enumerate(sls):
                    rows_v[b, i, sl] = r[j] * scale + (sm[j] + pe_regs[j])

        pltpu.sync_copy(pe_h, pe_all)
        issue_idx(0, 0)
        issue_idx(1, 1)
        wait_idx(0, 0)
        for r in range(_LD):
            compute_ci(0, r, r)
            issue_gather(0, r, r)

        def body(G, c):
            g0 = G * RPB
            for rr in range(RPB):
                g = g0 + rr
                b = rr % _NB
                wait_gather(b)
                compute(g, b)
                issue_out(g, b)

                if rr == 2:
                    wait_idx(2 * G + 1, 1)

                rrn = rr + _LD
                kbn, rn = divmod(rrn % RPB, _RB)
                bn = rrn % _NB
                if rrn == RPB:
                    @pl.when(g + _LD < S)
                    def _():
                        wait_idx(2 * G + 2, 0)

                @pl.when(g + _LD < S)
                def _():
                    compute_ci(kbn, rn, bn)

                    @pl.when(g >= 1)
                    def _():
                        wait_out_b(bn)

                    issue_gather(kbn, rn, bn)

                if rr == _RB - 1:
                    @pl.when((2 * G + 2) * _RB < S)
                    def _():
                        issue_idx(2 * G + 2, 0)
                if rr == RPB - 1:
                    @pl.when((2 * G + 3) * _RB < S)
                    def _():
                        issue_idx(2 * G + 3, 1)

            return c

        lax.fori_loop(0, S // RPB, body, 0)
        for b in range(_NB):
            wait_out_b(b)

    return _sc_kernel(src, mode, time, weekday, emb_loc, t2, pe)
